# baseline (speedup 1.0000x reference)
import functools

import jax
import jax.numpy as jnp
from jax import lax
from jax.experimental import pallas as pl
from jax.experimental.pallas import tpu as pltpu
from jax.experimental.pallas import tpu_sc as plsc

BATCH = 4096
NBHD = 200
VOCAB = 1000000
DIM = 64
LANES = 16
NVEC = DIM // LANES

CB = 4
VC = 512


@functools.cache
def _build_transpose_kernel():
    info = plsc.get_sparse_core_info()
    nw = info.num_cores * info.num_subcores
    n_main = (VOCAB // VC) // nw * nw
    tail = VOCAB - n_main * VC

    mesh = plsc.VectorSubcoreMesh(core_axis_name="c", subcore_axis_name="s")

    @functools.partial(
        pl.kernel,
        out_type=jax.ShapeDtypeStruct((VOCAB // 2, 2 * DIM), jnp.float32),
        scratch_types=[
            pltpu.VMEM((DIM, VC), jnp.float32),
            pltpu.VMEM((VC // 2, 2 * DIM), jnp.float32),
        ],
        mesh=mesh,
        compiler_params=pltpu.CompilerParams(
            use_tc_tiling_on_sc=True, needs_layout_passes=False
        ),
    )
    def k(tbl_t_hbm, tail_hbm, out_hbm, in_v, out_v):
        wid = lax.axis_index("s") * info.num_cores + lax.axis_index("c")

        def transpose_block(vbase, width):
            vbase = pl.multiple_of(vbase, 128)
            pltpu.sync_copy(
                tbl_t_hbm.at[:, pl.ds(vbase, width)], in_v.at[:, pl.ds(0, width)]
            )

            def jbody(j, carry):
                pos = lax.iota(jnp.int32, LANES) + j * LANES
                row16 = lax.shift_right_logical(pos, 1)
                colb16 = (pos & 1) * DIM
                for d in range(DIM):
                    x = in_v[d, pl.ds(j * LANES, LANES)]
                    plsc.store_scatter(out_v, [row16, colb16 + d], x)
                return carry

            lax.fori_loop(0, width // LANES, jbody, 0)
            pbase = pl.multiple_of(vbase // 2, 64)
            pltpu.sync_copy(
                out_v.at[pl.ds(0, width // 2)],
                out_hbm.at[pl.ds(pbase, width // 2)],
            )

        def chunk_body(c, carry):
            g = wid + c * nw
            transpose_block(g * VC, VC)
            return carry

        lax.fori_loop(0, n_main // nw, chunk_body, 0)

        n128 = tail // 128

        @pl.when(wid < n128)
        def _():
            transpose_block(n_main * VC + wid * 128, 128)

        @pl.when(wid == n128)
        def _():
            pltpu.sync_copy(
                tail_hbm, out_hbm.at[pl.ds((VOCAB - tail % 128) // 2, 32)]
            )

    return k


@functools.cache
def _build_gather_kernel():
    info = plsc.get_sparse_core_info()
    nw = info.num_cores * info.num_subcores
    rows_per_tile = BATCH // nw
    chunks = rows_per_tile // CB

    mesh = plsc.VectorSubcoreMesh(core_axis_name="c", subcore_axis_name="s")

    @functools.partial(
        pl.kernel,
        out_type=jax.ShapeDtypeStruct((BATCH, DIM), jnp.float32),
        scratch_types=[
            pltpu.VMEM((CB, NBHD), jnp.int32),
            pltpu.VMEM((CB, NBHD, DIM), jnp.float32),
            pltpu.VMEM((CB, DIM), jnp.float32),
            pltpu.SemaphoreType.DMA,
        ],
        mesh=mesh,
        compiler_params=pltpu.CompilerParams(use_tc_tiling_on_sc=False),
    )
    def k(idx_hbm, table_hbm, out_hbm, idx_v, rows_v, out_v, sem):
        wid = lax.axis_index("s") * info.num_cores + lax.axis_index("c")
        rbase = wid * rows_per_tile

        def chunk_body(c, carry):
            base = rbase + c * CB
            pltpu.sync_copy(idx_hbm.at[pl.ds(base, CB)], idx_v)
            cps = [
                pltpu.async_copy(table_hbm.at[idx_v.at[r]], rows_v.at[r], sem)
                for r in range(CB)
            ]
            for cp in cps:
                cp.wait()
            for r in range(CB):
                accs = tuple(jnp.zeros((LANES,), jnp.float32) for _ in range(NVEC))

                def body(n, a, r=r):
                    return tuple(
                        a[d] + rows_v[r, n, pl.ds(LANES * d, LANES)]
                        for d in range(NVEC)
                    )

                accs = lax.fori_loop(0, NBHD, body, accs)
                for d in range(NVEC):
                    out_v[r, pl.ds(LANES * d, LANES)] = accs[d] / float(NBHD)
            pltpu.sync_copy(out_v, out_hbm.at[pl.ds(base, CB)])
            return carry

        lax.fori_loop(0, chunks, chunk_body, 0)

    return k


def kernel(relation_indices, relation_table):
    idx = relation_indices[..., 0].astype(jnp.int32)
    tail = relation_table[VOCAB - 64 :, :].reshape(32, 2 * DIM)
    table_rm = _build_transpose_kernel()(relation_table.T, tail)
    table_lin = table_rm.reshape(VOCAB, DIM)
    return _build_gather_kernel()(idx, table_lin)


# speedup vs baseline: 3.7470x; 3.7470x over previous
import functools

import jax
import jax.numpy as jnp
from jax import lax
from jax.experimental import pallas as pl
from jax.experimental.pallas import tpu as pltpu
from jax.experimental.pallas import tpu_sc as plsc

BATCH = 4096
NBHD = 200
VOCAB = 1000000
DIM = 64
LANES = 16
NVEC = DIM // LANES

CB = 4
VC = 512


@functools.cache
def _build_transpose_kernel():
    info = plsc.get_sparse_core_info()
    nw = info.num_cores * info.num_subcores
    n_main = (VOCAB // VC) // nw * nw
    tail = VOCAB - n_main * VC

    mesh = plsc.VectorSubcoreMesh(core_axis_name="c", subcore_axis_name="s")

    @functools.partial(
        pl.kernel,
        out_type=jax.ShapeDtypeStruct((VOCAB * DIM,), jnp.float32),
        scratch_types=[
            pltpu.VMEM((DIM, VC), jnp.float32),
            pltpu.VMEM((VC * DIM,), jnp.float32),
        ],
        mesh=mesh,
        compiler_params=pltpu.CompilerParams(
            use_tc_tiling_on_sc=True, needs_layout_passes=False
        ),
    )
    def k(tbl_t_hbm, tail_hbm, out_hbm, in_v, out_v):
        wid = lax.axis_index("s") * info.num_cores + lax.axis_index("c")
        iota64 = lax.iota(jnp.int32, LANES) * DIM

        def transpose_block(vbase, width):
            vbase = pl.multiple_of(vbase, 128)
            pltpu.sync_copy(
                tbl_t_hbm.at[:, pl.ds(vbase, width)], in_v.at[:, pl.ds(0, width)]
            )

            @functools.partial(
                plsc.parallel_loop, 0, (width // LANES) * DIM, unroll=8
            )
            def _(t):
                j = lax.shift_right_logical(t, 6)
                d = t & (DIM - 1)
                x = in_v[d, pl.ds(j * LANES, LANES)]
                plsc.store_scatter(out_v, [iota64 + (j * (LANES * DIM) + d)], x)

            pbase = pl.multiple_of(vbase * DIM, 1024)
            pltpu.sync_copy(
                out_v.at[pl.ds(0, width * DIM)],
                out_hbm.at[pl.ds(pbase, width * DIM)],
            )

        def chunk_body(c, carry):
            g = wid + c * nw
            transpose_block(g * VC, VC)
            return carry

        lax.fori_loop(0, n_main // nw, chunk_body, 0)

        n128 = tail // 128

        @pl.when(wid < n128)
        def _():
            transpose_block(n_main * VC + wid * 128, 128)

        @pl.when(wid == n128)
        def _():
            pltpu.sync_copy(
                tail_hbm, out_hbm.at[pl.ds((VOCAB - tail % 128) * DIM, 64 * DIM)]
            )

    return k


@functools.cache
def _build_gather_kernel():
    info = plsc.get_sparse_core_info()
    nw = info.num_cores * info.num_subcores
    rows_per_tile = BATCH // nw
    chunks = rows_per_tile // CB

    mesh = plsc.VectorSubcoreMesh(core_axis_name="c", subcore_axis_name="s")

    @functools.partial(
        pl.kernel,
        out_type=jax.ShapeDtypeStruct((BATCH, DIM), jnp.float32),
        scratch_types=[
            pltpu.VMEM((CB, NBHD), jnp.int32),
            pltpu.VMEM((CB, NBHD, DIM), jnp.float32),
            pltpu.VMEM((CB, DIM), jnp.float32),
            pltpu.SemaphoreType.DMA,
        ],
        mesh=mesh,
        compiler_params=pltpu.CompilerParams(use_tc_tiling_on_sc=False),
    )
    def k(idx_hbm, table_hbm, out_hbm, idx_v, rows_v, out_v, sem):
        wid = lax.axis_index("s") * info.num_cores + lax.axis_index("c")
        rbase = wid * rows_per_tile

        def chunk_body(c, carry):
            base = rbase + c * CB
            pltpu.sync_copy(idx_hbm.at[pl.ds(base, CB)], idx_v)
            cps = [
                pltpu.async_copy(table_hbm.at[idx_v.at[r]], rows_v.at[r], sem)
                for r in range(CB)
            ]
            for cp in cps:
                cp.wait()
            for r in range(CB):
                accs = tuple(jnp.zeros((LANES,), jnp.float32) for _ in range(NVEC))

                def body(n, a, r=r):
                    return tuple(
                        a[d] + rows_v[r, n, pl.ds(LANES * d, LANES)]
                        for d in range(NVEC)
                    )

                accs = lax.fori_loop(0, NBHD, body, accs)
                for d in range(NVEC):
                    out_v[r, pl.ds(LANES * d, LANES)] = accs[d] / float(NBHD)
            pltpu.sync_copy(out_v, out_hbm.at[pl.ds(base, CB)])
            return carry

        lax.fori_loop(0, chunks, chunk_body, 0)

    return k


def kernel(relation_indices, relation_table):
    idx = relation_indices[..., 0].astype(jnp.int32)
    tail = relation_table[VOCAB - 64 :, :].reshape(-1)
    table_rm = _build_transpose_kernel()(relation_table.T, tail)
    table_lin = table_rm.reshape(VOCAB, DIM)
    return _build_gather_kernel()(idx, table_lin)


# speedup vs baseline: 4.4871x; 1.1975x over previous
import functools

import jax
import jax.numpy as jnp
from jax import lax
from jax.experimental import pallas as pl
from jax.experimental.pallas import tpu as pltpu
from jax.experimental.pallas import tpu_sc as plsc

BATCH = 4096
NBHD = 200
VOCAB = 1000000
DIM = 64
LANES = 16
NVEC = DIM // LANES

CB = 4
VC = 512


@functools.cache
def _build_transpose_kernel():
    info = plsc.get_sparse_core_info()
    nw = info.num_cores * info.num_subcores
    n_main = (VOCAB // VC) // nw * nw
    tail = VOCAB - n_main * VC

    mesh = plsc.VectorSubcoreMesh(core_axis_name="c", subcore_axis_name="s")

    @functools.partial(
        pl.kernel,
        out_type=jax.ShapeDtypeStruct((VOCAB * DIM,), jnp.float32),
        scratch_types=[
            pltpu.VMEM((DIM, VC), jnp.float32),
            pltpu.VMEM((VC * DIM,), jnp.float32),
        ],
        mesh=mesh,
        compiler_params=pltpu.CompilerParams(
            use_tc_tiling_on_sc=True, needs_layout_passes=False
        ),
    )
    def k(tbl_t_hbm, tail_hbm, out_hbm, in_v, out_v):
        wid = lax.axis_index("s") * info.num_cores + lax.axis_index("c")
        iota64 = lax.iota(jnp.int32, LANES) * DIM

        def transpose_block(vbase, width):
            vbase = pl.multiple_of(vbase, 128)
            pltpu.sync_copy(
                tbl_t_hbm.at[:, pl.ds(vbase, width)], in_v.at[:, pl.ds(0, width)]
            )

            @functools.partial(
                plsc.parallel_loop, 0, (width // LANES) * DIM, unroll=8
            )
            def _(t):
                j = lax.shift_right_logical(t, 6)
                d = t & (DIM - 1)
                x = in_v[d, pl.ds(j * LANES, LANES)]
                plsc.store_scatter(out_v, [iota64 + (j * (LANES * DIM) + d)], x)

            pbase = pl.multiple_of(vbase * DIM, 1024)
            pltpu.sync_copy(
                out_v.at[pl.ds(0, width * DIM)],
                out_hbm.at[pl.ds(pbase, width * DIM)],
            )

        def chunk_body(c, carry):
            g = wid + c * nw
            transpose_block(g * VC, VC)
            return carry

        lax.fori_loop(0, n_main // nw, chunk_body, 0)

        n128 = tail // 128

        @pl.when(wid < n128)
        def _():
            transpose_block(n_main * VC + wid * 128, 128)

        @pl.when(wid == n128)
        def _():
            pltpu.sync_copy(
                tail_hbm, out_hbm.at[pl.ds((VOCAB - tail % 128) * DIM, 64 * DIM)]
            )

    return k


@functools.cache
def _build_gather_kernel():
    info = plsc.get_sparse_core_info()
    nw = info.num_cores * info.num_subcores
    rows_per_tile = BATCH // nw
    chunks = rows_per_tile // CB

    mesh = plsc.VectorSubcoreMesh(core_axis_name="c", subcore_axis_name="s")

    @functools.partial(
        pl.kernel,
        out_type=jax.ShapeDtypeStruct((BATCH, DIM), jnp.float32),
        scratch_types=[
            pltpu.VMEM((2, CB, NBHD), jnp.int32),
            pltpu.VMEM((2, CB, NBHD, DIM), jnp.float32),
            pltpu.VMEM((CB, DIM), jnp.float32),
            pltpu.SemaphoreType.DMA,
            pltpu.SemaphoreType.DMA,
        ],
        mesh=mesh,
        compiler_params=pltpu.CompilerParams(use_tc_tiling_on_sc=False),
    )
    def k(idx_hbm, table_hbm, out_hbm, idx_v, rows_v, out_v, sem0, sem1):
        wid = lax.axis_index("s") * info.num_cores + lax.axis_index("c")
        rbase = wid * rows_per_tile
        sems = (sem0, sem1)

        def stage_and_fire(buf, c):
            base = rbase + c * CB
            pltpu.sync_copy(idx_hbm.at[pl.ds(base, CB)], idx_v.at[buf])
            return [
                pltpu.async_copy(
                    table_hbm.at[idx_v.at[buf, r]], rows_v.at[buf, r], sems[buf]
                )
                for r in range(CB)
            ]

        def compute_store(buf, c):
            for r in range(CB):
                accs = tuple(jnp.zeros((LANES,), jnp.float32) for _ in range(NVEC))

                def body(n, a, r=r):
                    return tuple(
                        a[d] + rows_v[buf, r, n, pl.ds(LANES * d, LANES)]
                        for d in range(NVEC)
                    )

                accs = lax.fori_loop(0, NBHD, body, accs)
                for d in range(NVEC):
                    out_v[r, pl.ds(LANES * d, LANES)] = accs[d] / float(NBHD)
            pltpu.sync_copy(out_v, out_hbm.at[pl.ds(rbase + c * CB, CB)])

        pending = stage_and_fire(0, 0)
        for c in range(1, chunks):
            for cp in pending:
                cp.wait()
            pending = stage_and_fire(c % 2, c)
            compute_store((c - 1) % 2, c - 1)
        for cp in pending:
            cp.wait()
        compute_store((chunks - 1) % 2, chunks - 1)

    return k


def kernel(relation_indices, relation_table):
    idx = relation_indices[..., 0].astype(jnp.int32)
    tail = relation_table[VOCAB - 64 :, :].reshape(-1)
    table_rm = _build_transpose_kernel()(relation_table.T, tail)
    table_lin = table_rm.reshape(VOCAB, DIM)
    return _build_gather_kernel()(idx, table_lin)
